# baseline (speedup 1.0000x reference)
import jax
import jax.numpy as jnp
import numpy as np
from jax import lax
from jax.experimental import pallas as pl
from jax.experimental.pallas import tpu as pltpu

_HID = 7
_NL = 10
_SLOPE = 0.01
_LANES = 128
_GROUP = 32
_K = 256
_TRI = 2048


def _leaky(a):
    return jnp.maximum(a, _SLOPE * a)


def _mlp_kernel(ws_ref, wg_ref, w9_ref, x_ref, o_ref):
    x0 = x_ref[0]
    x1 = x_ref[1]
    hs = [
        _leaky(jnp.dot(x0, ws_ref[p], preferred_element_type=jnp.float32)
               + jnp.dot(x1, ws_ref[4 + p], preferred_element_type=jnp.float32))
        for p in range(4)
    ]
    for l in range(1, _NL - 1):
        hs = [_leaky(jnp.dot(hp, wg_ref[l - 1],
                             preferred_element_type=jnp.float32))
              for hp in hs]
    acc = jnp.dot(hs[0], w9_ref[0], preferred_element_type=jnp.float32)
    for p in range(1, 4):
        acc = acc + jnp.dot(hs[p], w9_ref[p],
                            preferred_element_type=jnp.float32)
    o_ref[...] = _leaky(acc)


def _pack_gains(w_stack):
    eye = jnp.eye(_GROUP, dtype=jnp.float32)

    def blockdiag(blk):
        a, b = blk.shape
        m = eye[:, None, :, None] * blk[None, :, None, :]
        return m.reshape(_GROUP * a, _GROUP * b)

    ws_parts = []
    for f in range(2):
        band = blockdiag(w_stack[0, f:f + 1, :8])
        ws_parts.extend(
            jnp.pad(band, ((_GROUP * p, 96 - _GROUP * p), (0, 0)))
            for p in range(4))
    ws = jnp.stack(ws_parts)
    g_l = [
        blockdiag(w_stack[l, :8, :8])
        for l in range(1, _NL - 1)
    ]
    wg = jnp.stack(g_l)

    band9 = blockdiag(w_stack[_NL - 1, :8, :1])
    w9 = jnp.stack([
        jnp.pad(band9, ((0, 0), (_GROUP * p, 96 - _GROUP * p)))
        for p in range(4)
    ])
    return ws, wg, w9


def _deep_mlp(x, w_stack):
    B, in_f = x.shape
    ws, wg, w9 = _pack_gains(w_stack)
    rows = B // _LANES
    xr = x.T.reshape(2, rows, _LANES)
    out = pl.pallas_call(
        _mlp_kernel,
        out_shape=jax.ShapeDtypeStruct((rows, _LANES), jnp.float32),
        grid=(rows // _TRI,),
        in_specs=[
            pl.BlockSpec((8, _LANES, _K), lambda b: (0, 0, 0)),
            pl.BlockSpec((8, _K, _K), lambda b: (0, 0, 0)),
            pl.BlockSpec((4, _K, _LANES), lambda b: (0, 0, 0)),
            pl.BlockSpec((2, _TRI, _LANES), lambda b: (0, b, 0)),
        ],
        out_specs=pl.BlockSpec((_TRI, _LANES), lambda b: (b, 0)),
        compiler_params=pltpu.CompilerParams(
            dimension_semantics=("arbitrary",),
            vmem_limit_bytes=56 * 1024 * 1024,
        ),
    )(ws, wg, w9, xr)
    return out.reshape(B, 1)


def kernel(x, w_stack):
    return _deep_mlp(x, w_stack)


# speedup vs baseline: 1.0631x; 1.0631x over previous
import jax
import jax.numpy as jnp
import numpy as np
from jax import lax
from jax.experimental import pallas as pl
from jax.experimental.pallas import tpu as pltpu

_HID = 7
_NL = 10
_SLOPE = 0.01
_LANES = 128
_GROUP = 32
_K = 256
_TRI = 2048


def _leaky(a):
    return jnp.maximum(a, _SLOPE * a)


def _mlp_kernel(ws_ref, wg_ref, w9_ref, x_ref, o_ref):
    x0 = x_ref[0]
    x1 = x_ref[1]
    hs = [
        _leaky(jnp.dot(x0, ws_ref[p], preferred_element_type=jnp.float32)
               + jnp.dot(x1, ws_ref[4 + p], preferred_element_type=jnp.float32))
        .astype(jnp.bfloat16)
        for p in range(4)
    ]
    h = jnp.concatenate(hs, axis=0)
    for l in range(1, _NL - 1):
        h = _leaky(jnp.dot(h, wg_ref[l - 1],
                           preferred_element_type=jnp.float32)
                   ).astype(jnp.bfloat16)
    acc = jnp.dot(h[0:_TRI], w9_ref[0], preferred_element_type=jnp.float32)
    for p in range(1, 4):
        acc = acc + jnp.dot(h[p * _TRI:(p + 1) * _TRI], w9_ref[p],
                            preferred_element_type=jnp.float32)
    o_ref[...] = _leaky(acc)


def _pack_gains(w_stack):
    eye = jnp.eye(_GROUP, dtype=jnp.float32)

    def blockdiag(blk):
        a, b = blk.shape
        m = eye[:, None, :, None] * blk[None, :, None, :]
        return m.reshape(_GROUP * a, _GROUP * b)

    ws_parts = []
    for f in range(2):
        band = blockdiag(w_stack[0, f:f + 1, :8])
        ws_parts.extend(
            jnp.pad(band, ((_GROUP * p, 96 - _GROUP * p), (0, 0)))
            for p in range(4))
    ws = jnp.stack(ws_parts)
    g_l = [
        blockdiag(w_stack[l, :8, :8])
        for l in range(1, _NL - 1)
    ]
    wg = jnp.stack(g_l)

    band9 = blockdiag(w_stack[_NL - 1, :8, :1])
    w9 = jnp.stack([
        jnp.pad(band9, ((0, 0), (_GROUP * p, 96 - _GROUP * p)))
        for p in range(4)
    ])
    return (ws.astype(jnp.bfloat16), wg.astype(jnp.bfloat16),
            w9.astype(jnp.bfloat16))


def _deep_mlp(x, w_stack):
    B, in_f = x.shape
    ws, wg, w9 = _pack_gains(w_stack)
    rows = B // _LANES
    xr = x.T.reshape(2, rows, _LANES).astype(jnp.bfloat16)
    out = pl.pallas_call(
        _mlp_kernel,
        out_shape=jax.ShapeDtypeStruct((rows, _LANES), jnp.float32),
        grid=(rows // _TRI,),
        in_specs=[
            pl.BlockSpec((8, _LANES, _K), lambda b: (0, 0, 0)),
            pl.BlockSpec((8, _K, _K), lambda b: (0, 0, 0)),
            pl.BlockSpec((4, _K, _LANES), lambda b: (0, 0, 0)),
            pl.BlockSpec((2, _TRI, _LANES), lambda b: (0, b, 0)),
        ],
        out_specs=pl.BlockSpec((_TRI, _LANES), lambda b: (b, 0)),
        compiler_params=pltpu.CompilerParams(
            dimension_semantics=("arbitrary",),
            vmem_limit_bytes=56 * 1024 * 1024,
        ),
    )(ws, wg, w9, xr)
    return out.reshape(B, 1)


def kernel(x, w_stack):
    return _deep_mlp(x, w_stack)
